# Initial kernel scaffold; baseline (speedup 1.0000x reference)
import jax
import jax.numpy as jnp
from jax.experimental import pallas as pl


def kernel(x, edge_index, batch, lap_enc, edge_attr, W_h, b_h, W_lap, b_lap, W_e, b_e, A, B, C, U, V, bA, bB, bC, bU, bV, W1, b1, W2, b2):
    raise NotImplementedError("write your pallas kernel here")


# speedup vs baseline: 1.5168x; 1.5168x over previous
import functools

import jax
import jax.numpy as jnp
from jax import lax
from jax.experimental import pallas as pl
from jax.experimental.pallas import tpu as pltpu
from jax.experimental.pallas import tpu_sc as plsc

N, E, D, G, L = 10000, 160000, 166, 128, 3
P = 176
P2 = 352
PM = 192
HW = 96
INV_D = 1.0 / float(D)

NC, NS = 2, 16
NW = NC * NS
CH = 128
EW = 4992
NCHUNK = EW // CH
EXTRA_BASE = NW * EW
NPW = 312
PCH = 104
POOL_TAIL = NW * NPW
NROWS_TILE = N // NS

_mesh = plsc.VectorSubcoreMesh(core_axis_name="c", subcore_axis_name="s")
_sc_params = pltpu.CompilerParams(use_tc_tiling_on_sc=False)


def _pad2(w, r, c):
  return jnp.pad(w, ((0, r - w.shape[0]), (0, c - w.shape[1])))


def _pad1(b, c):
  return jnp.pad(b, (0, c - b.shape[0])).reshape(1, c)


def _mask_row():
  return (lax.broadcasted_iota(jnp.int32, (1, P), 1) < D).astype(jnp.float32)


def _ln_masked(v, mask):
  m = jnp.sum(v, axis=-1, keepdims=True) * INV_D
  d = (v - m) * mask
  var = jnp.sum(d * d, axis=-1, keepdims=True) * INV_D
  return d * lax.rsqrt(var + 1e-5)


def _mm_bias_body(x_ref, w_ref, b_ref, o_ref):
  o_ref[...] = jnp.dot(x_ref[...], w_ref[...],
                       preferred_element_type=jnp.float32) + b_ref[...]


def _tc_mm_bias(x, w, b, blk):
  rows = x.shape[0]
  return pl.pallas_call(
      _mm_bias_body,
      grid=(rows // blk,),
      in_specs=[
          pl.BlockSpec((blk, x.shape[1]), lambda i: (i, 0)),
          pl.BlockSpec(w.shape, lambda i: (0, 0)),
          pl.BlockSpec(b.shape, lambda i: (0, 0)),
      ],
      out_specs=pl.BlockSpec((blk, w.shape[1]), lambda i: (i, 0)),
      out_shape=jax.ShapeDtypeStruct((rows, w.shape[1]), jnp.float32),
  )(x, w, b)


def _edge_body(e_ref, gab_ref, gv_ref, c_ref, bc_ref, en_ref, msg_ref,
               sig_ref):
  mask = _mask_row()
  e = e_ref[...]
  e_hat = (jnp.dot(e, c_ref[...], preferred_element_type=jnp.float32)
           + bc_ref[...] + gab_ref[...])
  sigma = jax.nn.sigmoid(e_hat)
  zpad = jnp.zeros((e.shape[0], PM - P), jnp.float32)
  sig_ref[...] = jnp.concatenate([sigma, zpad], axis=1)
  msg_ref[...] = jnp.concatenate([sigma * gv_ref[...], zpad], axis=1)
  en_ref[...] = _ln_masked(e + e_hat, mask)


def _tc_edge(e, gab, gv, C, bC, blk=1000):
  nb = E // blk
  spec = pl.BlockSpec((blk, P), lambda i: (i, 0))
  mspec = pl.BlockSpec((blk, PM), lambda i: (i, 0))
  out = jax.ShapeDtypeStruct((E, P), jnp.float32)
  mout = jax.ShapeDtypeStruct((E, PM), jnp.float32)
  return pl.pallas_call(
      _edge_body,
      grid=(nb,),
      in_specs=[spec, spec, spec,
                pl.BlockSpec((P, P), lambda i: (0, 0)),
                pl.BlockSpec((1, P), lambda i: (0, 0))],
      out_specs=[spec, mspec, mspec],
      out_shape=[out, mout, mout],
  )(e, gab, gv, C, bC)


def _update_body(h_ref, hu_ref, np_ref, dp_ref, w1_ref, b1_ref, w2_ref,
                 b2_ref, o_ref):
  mask = _mask_row()
  num = (np_ref[0] + np_ref[1])[:, :P]
  den = (dp_ref[0] + dp_ref[1])[:, :P] + 1e-6
  h = _ln_masked(h_ref[...] + hu_ref[...] + num / den, mask)
  hid = jax.nn.relu(jnp.dot(h, w1_ref[...],
                            preferred_element_type=jnp.float32) + b1_ref[...])
  ffn = jnp.dot(hid, w2_ref[...],
                preferred_element_type=jnp.float32) + b2_ref[...]
  o_ref[...] = _ln_masked(h + ffn, mask)


def _tc_update(h, hu, num_p, den_p, W1, b1, W2, b2, blk=1000):
  nb = N // blk
  spec = pl.BlockSpec((blk, P), lambda i: (i, 0))
  pspec = pl.BlockSpec((NC, blk, PM), lambda i: (0, i, 0))
  return pl.pallas_call(
      _update_body,
      grid=(nb,),
      in_specs=[spec, spec, pspec, pspec,
                pl.BlockSpec((P, P2), lambda i: (0, 0)),
                pl.BlockSpec((1, P2), lambda i: (0, 0)),
                pl.BlockSpec((P2, P), lambda i: (0, 0)),
                pl.BlockSpec((1, P), lambda i: (0, 0))],
      out_specs=spec,
      out_shape=jax.ShapeDtypeStruct((N, P), jnp.float32),
  )(h, hu, num_p, den_p, W1, b1, W2, b2)


def _final_body(sp_ref, cp_ref, o_ref):
  s = sp_ref[0] + sp_ref[1]
  c = cp_ref[0, :, 0:1] + cp_ref[1, :, 0:1]
  o_ref[...] = (s / jnp.maximum(c, 1.0))[:, :D]


def _tc_final(sums_p, cnt_p):
  return pl.pallas_call(
      _final_body,
      in_specs=[pl.BlockSpec((NC, G, P), lambda: (0, 0, 0)),
                pl.BlockSpec((NC, G, 16), lambda: (0, 0, 0))],
      out_specs=pl.BlockSpec((G, D), lambda: (0, 0)),
      out_shape=jax.ShapeDtypeStruct((G, D), jnp.float32),
  )(sums_p, cnt_p)


def _gather_body(hA, hB, hV, src_h, dst_h, gab_h, gv_h,
                 idx_d, idx_s, bufA, bufB, semA, semB):
  cid = lax.axis_index("c")
  sid = lax.axis_index("s")
  wid = sid * NC + cid
  base = wid * EW

  def add_rows(r, _):
    for j in range(P // 16):
      sl = pl.ds(j * 16, 16)
      bufA[r, sl] = bufA[r, sl] + bufB[r, sl]
    return 0

  def do_chunk(off):
    pltpu.sync_copy(dst_h.at[pl.ds(off, CH)], idx_d)
    pltpu.sync_copy(src_h.at[pl.ds(off, CH)], idx_s)
    cpA = pltpu.async_copy(hA.at[idx_d], bufA, semA)
    cpB = pltpu.async_copy(hB.at[idx_s], bufB, semB)
    cpA.wait()
    cpB.wait()
    lax.fori_loop(0, CH, add_rows, 0)
    pltpu.sync_copy(bufA, gab_h.at[pl.ds(off, CH), :])
    pltpu.async_copy(hV.at[idx_s], bufB, semB).wait()
    pltpu.sync_copy(bufB, gv_h.at[pl.ds(off, CH), :])

  def loop(i, _):
    do_chunk(base + i * CH)
    return 0

  lax.fori_loop(0, NCHUNK, loop, 0)

  @pl.when(wid < 2)
  def _():
    do_chunk(EXTRA_BASE + wid * CH)


def _sc_gather(hA, hB, hV, src, dst):
  out = jax.ShapeDtypeStruct((E, P), jnp.float32)
  k = pl.kernel(
      _gather_body,
      out_type=[out, out],
      mesh=_mesh,
      compiler_params=_sc_params,
      scratch_types=[
          pltpu.VMEM((CH,), jnp.int32),
          pltpu.VMEM((CH,), jnp.int32),
          pltpu.VMEM((CH, P), jnp.float32),
          pltpu.VMEM((CH, P), jnp.float32),
          pltpu.SemaphoreType.DMA,
          pltpu.SemaphoreType.DMA,
      ],
  )
  return k(hA, hB, hV, src, dst)


def _scatter_body(msg_h, sig_h, dst_h, zeros_h, num_h, den_h,
                  idx_v, dbuf, acc):
  cid = lax.axis_index("c")
  sid = lax.axis_index("s")
  wid = sid * NC + cid
  base = wid * EW
  slab = pl.ds(sid * NROWS_TILE, NROWS_TILE)

  def run_phase(data_h, out_h, c0):
    pltpu.sync_copy(zeros_h.at[:, pl.ds(0, HW)], acc.at[slab])
    plsc.subcore_barrier()

    def do_chunk(off):
      pltpu.sync_copy(dst_h.at[pl.ds(off, CH)], idx_v)
      pltpu.sync_copy(data_h.at[pl.ds(off, CH), pl.ds(c0, HW)], dbuf)
      pltpu.sync_copy(dbuf, acc.at[idx_v], add=True)

    def loop(i, _):
      do_chunk(base + i * CH)
      return 0

    lax.fori_loop(0, NCHUNK, loop, 0)

    @pl.when(wid < 2)
    def _():
      do_chunk(EXTRA_BASE + wid * CH)

    plsc.subcore_barrier()
    pltpu.sync_copy(acc.at[slab], out_h.at[cid, slab, pl.ds(c0, HW)])
    plsc.subcore_barrier()

  run_phase(msg_h, num_h, 0)
  run_phase(msg_h, num_h, HW)
  run_phase(sig_h, den_h, 0)
  run_phase(sig_h, den_h, HW)


def _sc_scatter(msg, sig, dst, zeros):
  out = jax.ShapeDtypeStruct((NC, N, PM), jnp.float32)
  k = pl.kernel(
      _scatter_body,
      out_type=[out, out],
      mesh=_mesh,
      compiler_params=_sc_params,
      scratch_types=[
          pltpu.VMEM((CH,), jnp.int32),
          pltpu.VMEM((CH, HW), jnp.float32),
          pltpu.VMEM_SHARED((N, HW), jnp.float32),
      ],
  )
  return k(msg, sig, dst, zeros)


def _pool_body(h_h, batch_h, zeros_h, ones_h, sums_h, cnt_h,
               idx_v, idx_t, hbuf, ones_v, acc_s, acc_c):
  cid = lax.axis_index("c")
  sid = lax.axis_index("s")
  wid = sid * NC + cid
  base = wid * NPW
  gslab = pl.ds(sid * (G // NS), G // NS)

  pltpu.sync_copy(ones_h, ones_v)
  pltpu.sync_copy(zeros_h.at[pl.ds(0, G // NS), :], acc_s.at[gslab])
  pltpu.sync_copy(zeros_h.at[pl.ds(0, G // NS), pl.ds(0, 16)],
                  acc_c.at[gslab])
  plsc.subcore_barrier()

  def do_chunk(i, _):
    off = base + i * PCH
    pltpu.sync_copy(batch_h.at[pl.ds(off, PCH)], idx_v)
    pltpu.sync_copy(h_h.at[pl.ds(off, PCH), :], hbuf)
    pltpu.sync_copy(hbuf, acc_s.at[idx_v], add=True)
    pltpu.sync_copy(ones_v, acc_c.at[idx_v], add=True)
    return 0

  lax.fori_loop(0, NPW // PCH, do_chunk, 0)

  @pl.when(wid == 0)
  def _():
    pltpu.sync_copy(batch_h.at[pl.ds(POOL_TAIL, 16)], idx_t)
    pltpu.sync_copy(h_h.at[pl.ds(POOL_TAIL, 16), :], hbuf.at[pl.ds(0, 16), :])
    pltpu.sync_copy(hbuf.at[pl.ds(0, 16), :], acc_s.at[idx_t], add=True)
    pltpu.sync_copy(ones_v.at[pl.ds(0, 16), :], acc_c.at[idx_t], add=True)

  plsc.subcore_barrier()
  pltpu.sync_copy(acc_s.at[gslab], sums_h.at[cid, gslab, :])
  pltpu.sync_copy(acc_c.at[gslab], cnt_h.at[cid, gslab, :])


def _sc_pool(h, batch, zeros, ones):
  k = pl.kernel(
      _pool_body,
      out_type=[jax.ShapeDtypeStruct((NC, G, P), jnp.float32),
                jax.ShapeDtypeStruct((NC, G, 16), jnp.float32)],
      mesh=_mesh,
      compiler_params=_sc_params,
      scratch_types=[
          pltpu.VMEM((PCH,), jnp.int32),
          pltpu.VMEM((16,), jnp.int32),
          pltpu.VMEM((PCH, P), jnp.float32),
          pltpu.VMEM((PCH, 16), jnp.float32),
          pltpu.VMEM_SHARED((G, P), jnp.float32),
          pltpu.VMEM_SHARED((G, 16), jnp.float32),
      ],
  )
  return k(h, batch, zeros, ones)


def kernel(x, edge_index, batch, lap_enc, edge_attr, W_h, b_h, W_lap, b_lap,
           W_e, b_e, A, B, C, U, V, bA, bB, bC, bU, bV, W1, b1, W2, b2):
  src = edge_index[0].astype(jnp.int32)
  dst = edge_index[1].astype(jnp.int32)
  batch = batch.astype(jnp.int32)

  W_in = jnp.concatenate([_pad2(W_h, 21, P), _pad2(W_lap, 8, P)], axis=0)
  b_in = _pad1(b_h + b_lap, P)
  W_e_p = _pad2(W_e, 16, P)
  b_e_p = _pad1(b_e, P)
  Wn = [jnp.concatenate([_pad2(A[l], P, P), _pad2(B[l], P, P),
                         _pad2(V[l], P, P), _pad2(U[l], P, P)], axis=1)
        for l in range(L)]
  bn = [jnp.concatenate([_pad1(bA[l], P), _pad1(bB[l], P),
                         _pad1(bV[l], P), _pad1(bU[l], P)], axis=1)
        for l in range(L)]
  C_p = [_pad2(C[l], P, P) for l in range(L)]
  bC_p = [_pad1(bC[l], P) for l in range(L)]
  W1_p = [_pad2(W1[l], P, P2) for l in range(L)]
  b1_p = [_pad1(b1[l], P2) for l in range(L)]
  W2_p = [_pad2(W2[l], P2, P) for l in range(L)]
  b2_p = [_pad1(b2[l], P) for l in range(L)]

  zeros_c = jnp.zeros((NROWS_TILE, P), jnp.float32)
  ones_c = jnp.ones((PCH, 16), jnp.float32)

  xl = jnp.concatenate([x, lap_enc], axis=1)
  h = _tc_mm_bias(xl, W_in, b_in, blk=1000)
  e = _tc_mm_bias(edge_attr, W_e_p, b_e_p, blk=1000)

  for l in range(L):
    hm = _tc_mm_bias(h, Wn[l], bn[l], blk=1000)
    hA = hm[:, 0:P]
    hB = hm[:, P:2 * P]
    hV = hm[:, 2 * P:3 * P]
    hU = hm[:, 3 * P:4 * P]
    gab, gv = _sc_gather(hA, hB, hV, src, dst)
    e, msg, sig = _tc_edge(e, gab, gv, C_p[l], bC_p[l])
    num_p, den_p = _sc_scatter(msg, sig, dst, zeros_c)
    h = _tc_update(h, hU, num_p, den_p, W1_p[l], b1_p[l], W2_p[l], b2_p[l])

  sums_p, cnt_p = _sc_pool(h, batch, zeros_c, ones_c)
  return _tc_final(sums_p, cnt_p)
